# baseline (speedup 1.0000x reference)
import jax
import jax.numpy as jnp
from jax.experimental import pallas as pl
from jax.experimental.pallas import tpu as pltpu

_TT = 256
_EP = 128

_INTERPRET = jax.default_backend() == "cpu"


def _gate(x, wg, bg, lane_dim):
    scores = jnp.dot(x, wg, preferred_element_type=jnp.float32) + bg
    lane = jax.lax.broadcasted_iota(jnp.int32, scores.shape, 1)
    mask = lane < lane_dim
    neg = jnp.float32(-jnp.inf)
    s = jnp.where(mask, scores, neg)
    m = jnp.max(s, axis=-1, keepdims=True)
    ex = jnp.where(mask, jnp.exp(s - m), 0.0)
    p = ex / jnp.sum(ex, axis=-1, keepdims=True)
    pm = jnp.where(mask, p, neg)
    m0 = jnp.max(pm, axis=-1, keepdims=True)
    i0 = jnp.min(jnp.where(pm == m0, lane, _EP), axis=-1, keepdims=True)
    pm1 = jnp.where(lane == i0, neg, pm)
    m1 = jnp.max(pm1, axis=-1, keepdims=True)
    i1 = jnp.min(jnp.where(pm1 == m1, lane, _EP), axis=-1, keepdims=True)
    r = jnp.exp(m1 - m0)
    w0 = 1.0 / (1.0 + r)
    w1 = r / (1.0 + r)
    return p, i0, w0, i1, w1


def _moe_dense_body(n_tok, n_exp,
                    x_ref, wg_ref, bg_ref, w1_ref, b1_ref, w2_ref, b2_ref,
                    out_ref, aux_ref, imp_ref, load_ref):
    tt = pl.program_id(0)
    e = pl.program_id(1)
    nt = pl.num_programs(0)
    ne = pl.num_programs(1)
    x = x_ref[...]
    p, i0, w0, i1, w1 = _gate(x, wg_ref[...], bg_ref[...], n_exp)
    c = jnp.where(i0 == e, w0, 0.0) + jnp.where(i1 == e, w1, 0.0)
    h = jnp.maximum(
        jnp.dot(x, w1_ref[0], preferred_element_type=jnp.float32) + b1_ref[0],
        0.0)
    y = jnp.dot(h, w2_ref[0], preferred_element_type=jnp.float32) + b2_ref[0]
    contrib = c * y

    @pl.when(e == 0)
    def _():
        out_ref[...] = contrib
        psum = jnp.sum(p, axis=0, keepdims=True)
        lsum = jnp.sum((p > 0).astype(jnp.float32), axis=0, keepdims=True)

        @pl.when(tt == 0)
        def _():
            imp_ref[...] = psum
            load_ref[...] = lsum

        @pl.when(tt != 0)
        def _():
            imp_ref[...] += psum
            load_ref[...] += lsum

    @pl.when(e != 0)
    def _():
        out_ref[...] += contrib

    @pl.when((tt == nt - 1) & (e == ne - 1))
    def _():
        tf = jnp.float32(n_tok)
        aux = jnp.sum(
            (imp_ref[...] / tf) * (load_ref[...] / tf),
            axis=-1, keepdims=True) * jnp.float32(n_exp)
        aux_ref[...] = aux


def kernel(x, Wg, bg, W1, b1, W2, b2):
    B, S, D = x.shape
    E = Wg.shape[1]
    H = W1.shape[2]
    T = B * S
    xf = x.reshape(T, D)
    wgp = jnp.zeros((D, _EP), jnp.float32).at[:, :E].set(Wg)
    bgp = jnp.zeros((1, _EP), jnp.float32).at[0, :E].set(bg)
    b1r = b1.reshape(E, 1, H)
    b2r = b2.reshape(E, 1, D)

    import functools
    body = functools.partial(_moe_dense_body, T, E)
    out, aux = pl.pallas_call(
        body,
        grid=(T // _TT, E),
        in_specs=[
            pl.BlockSpec((_TT, D), lambda tt, e: (tt, 0)),
            pl.BlockSpec((D, _EP), lambda tt, e: (0, 0)),
            pl.BlockSpec((1, _EP), lambda tt, e: (0, 0)),
            pl.BlockSpec((1, D, H), lambda tt, e: (e, 0, 0)),
            pl.BlockSpec((1, 1, H), lambda tt, e: (e, 0, 0)),
            pl.BlockSpec((1, H, D), lambda tt, e: (e, 0, 0)),
            pl.BlockSpec((1, 1, D), lambda tt, e: (e, 0, 0)),
        ],
        out_specs=[
            pl.BlockSpec((_TT, D), lambda tt, e: (tt, 0)),
            pl.BlockSpec((1, 1), lambda tt, e: (0, 0)),
        ],
        out_shape=[
            jax.ShapeDtypeStruct((T, D), jnp.float32),
            jax.ShapeDtypeStruct((1, 1), jnp.float32),
        ],
        scratch_shapes=[
            pltpu.VMEM((1, _EP), jnp.float32),
            pltpu.VMEM((1, _EP), jnp.float32),
        ],
        compiler_params=pltpu.CompilerParams(
            dimension_semantics=("arbitrary", "arbitrary")),
        interpret=_INTERPRET,
    )(xf, wgp, bgp, W1, b1r, W2, b2r)
    return out.reshape(B, S, D), aux[0, 0]


# speedup vs baseline: 2.2344x; 2.2344x over previous
import functools

import jax
import jax.numpy as jnp
from jax.experimental import pallas as pl
from jax.experimental.pallas import tpu as pltpu

_TT = 256
_EP = 128

_INTERPRET = jax.default_backend() == "cpu"


def _gate(x, wg, bg, n_exp):
    scores = jnp.dot(x, wg, preferred_element_type=jnp.float32) + bg
    lane = jax.lax.broadcasted_iota(jnp.int32, scores.shape, 1)
    mask = lane < n_exp
    neg = jnp.float32(-jnp.inf)
    s = jnp.where(mask, scores, neg)
    m = jnp.max(s, axis=-1, keepdims=True)
    ex = jnp.where(mask, jnp.exp(s - m), 0.0)
    p = ex / jnp.sum(ex, axis=-1, keepdims=True)
    pm = jnp.where(mask, p, neg)
    m0 = jnp.max(pm, axis=-1, keepdims=True)
    i0 = jnp.min(jnp.where(pm == m0, lane, _EP), axis=-1, keepdims=True)
    pm1 = jnp.where(lane == i0, neg, pm)
    m1 = jnp.max(pm1, axis=-1, keepdims=True)
    i1 = jnp.min(jnp.where(pm1 == m1, lane, _EP), axis=-1, keepdims=True)
    r = jnp.exp(m1 - m0)
    w0 = 1.0 / (1.0 + r)
    w1 = r / (1.0 + r)
    return p, i0, w0, i1, w1


def _moe_dense_body(n_tok, n_exp,
                    x_ref, wg_ref, bg_ref, w1_ref, b1_ref, w2_ref, b2_ref,
                    out_ref, aux_ref, imp_ref, load_ref):
    tt = pl.program_id(0)
    nt = pl.num_programs(0)
    x = x_ref[...]
    p, i0, w0, i1, w1 = _gate(x, wg_ref[...], bg_ref[...], n_exp)

    acc = jnp.zeros(out_ref.shape, jnp.float32)
    for e in range(n_exp):
        c = jnp.where(i0 == e, w0, 0.0) + jnp.where(i1 == e, w1, 0.0)
        h = jnp.maximum(
            jnp.dot(x, w1_ref[e], preferred_element_type=jnp.float32)
            + b1_ref[e], 0.0)
        y = (jnp.dot(h, w2_ref[e], preferred_element_type=jnp.float32)
             + b2_ref[e])
        acc += c * y
    out_ref[...] = acc

    psum = jnp.sum(p, axis=0, keepdims=True)
    lsum = jnp.sum((p > 0).astype(jnp.float32), axis=0, keepdims=True)

    @pl.when(tt == 0)
    def _():
        imp_ref[...] = psum
        load_ref[...] = lsum

    @pl.when(tt != 0)
    def _():
        imp_ref[...] += psum
        load_ref[...] += lsum

    @pl.when(tt == nt - 1)
    def _():
        tf = jnp.float32(n_tok)
        aux_ref[...] = jnp.sum(
            (imp_ref[...] / tf) * (load_ref[...] / tf),
            axis=-1, keepdims=True) * jnp.float32(n_exp)


def kernel(x, Wg, bg, W1, b1, W2, b2):
    B, S, D = x.shape
    E = Wg.shape[1]
    H = W1.shape[2]
    T = B * S
    xf = x.reshape(T, D)
    wgp = jnp.zeros((D, _EP), jnp.float32).at[:, :E].set(Wg)
    bgp = jnp.zeros((1, _EP), jnp.float32).at[0, :E].set(bg)
    b1r = b1.reshape(E, 1, H)
    b2r = b2.reshape(E, 1, D)

    body = functools.partial(_moe_dense_body, T, E)
    out, aux = pl.pallas_call(
        body,
        grid=(T // _TT,),
        in_specs=[
            pl.BlockSpec((_TT, D), lambda tt: (tt, 0)),
            pl.BlockSpec((D, _EP), lambda tt: (0, 0)),
            pl.BlockSpec((1, _EP), lambda tt: (0, 0)),
            pl.BlockSpec((E, D, H), lambda tt: (0, 0, 0)),
            pl.BlockSpec((E, 1, H), lambda tt: (0, 0, 0)),
            pl.BlockSpec((E, H, D), lambda tt: (0, 0, 0)),
            pl.BlockSpec((E, 1, D), lambda tt: (0, 0, 0)),
        ],
        out_specs=[
            pl.BlockSpec((_TT, D), lambda tt: (tt, 0)),
            pl.BlockSpec((1, 1), lambda tt: (0, 0)),
        ],
        out_shape=[
            jax.ShapeDtypeStruct((T, D), jnp.float32),
            jax.ShapeDtypeStruct((1, 1), jnp.float32),
        ],
        scratch_shapes=[
            pltpu.VMEM((1, _EP), jnp.float32),
            pltpu.VMEM((1, _EP), jnp.float32),
        ],
        compiler_params=pltpu.CompilerParams(
            dimension_semantics=("arbitrary",)),
        interpret=_INTERPRET,
    )(xf, wgp, bgp, W1, b1r, W2, b2r)
    return out.reshape(B, S, D), aux[0, 0]
